# Initial kernel scaffold; baseline (speedup 1.0000x reference)
import jax
import jax.numpy as jnp
from jax.experimental import pallas as pl


def kernel(x, projection, centroids):
    raise NotImplementedError("write your pallas kernel here")


# speedup vs baseline: 1.1851x; 1.1851x over previous
import jax
import jax.numpy as jnp
from jax.experimental import pallas as pl
from jax.experimental.pallas import tpu as pltpu


def _centroid_norm_kernel(c_ref, cn_ref):
    c = c_ref[...]
    s = jnp.sum(c * c, axis=1, keepdims=True)
    inv = 1.0 / (jnp.sqrt(s) + 1e-12)
    cn_ref[...] = (c * inv).astype(jnp.bfloat16)


def _main_kernel(x_ref, p_ref, cn_ref, xhv_ref, preds_ref):
    xb = x_ref[...].astype(jnp.bfloat16)
    xh = jax.lax.dot_general(
        xb, p_ref[...], (((1,), (1,)), ((), ())),
        preferred_element_type=jnp.float32)
    xhv_ref[...] = xh
    s = jnp.sum(xh * xh, axis=1, keepdims=True)
    factor = 1.0 / (jnp.sqrt(s) + 1e-12)
    ph = jax.lax.dot_general(
        xh.astype(jnp.bfloat16), cn_ref[...], (((1,), (1,)), ((), ())),
        preferred_element_type=jnp.float32)
    preds_ref[...] = ph * factor


@jax.jit
def kernel(x, projection, centroids):
    B, F = x.shape
    D, _ = projection.shape
    C, _ = centroids.shape

    BC = 256
    cn = pl.pallas_call(
        _centroid_norm_kernel,
        grid=(C // BC,),
        in_specs=[pl.BlockSpec((BC, D), lambda i: (i, 0))],
        out_specs=pl.BlockSpec((BC, D), lambda i: (i, 0)),
        out_shape=jax.ShapeDtypeStruct((C, D), jnp.bfloat16),
    )(centroids)

    pbf = projection.astype(jnp.bfloat16)

    BI = 256
    xhv, preds = pl.pallas_call(
        _main_kernel,
        grid=(B // BI,),
        in_specs=[
            pl.BlockSpec((BI, F), lambda i: (i, 0)),
            pl.BlockSpec((D, F), lambda i: (0, 0)),
            pl.BlockSpec((C, D), lambda i: (0, 0)),
        ],
        out_specs=[
            pl.BlockSpec((BI, D), lambda i: (i, 0)),
            pl.BlockSpec((BI, C), lambda i: (i, 0)),
        ],
        out_shape=[
            jax.ShapeDtypeStruct((B, D), jnp.float32),
            jax.ShapeDtypeStruct((B, C), jnp.float32),
        ],
        compiler_params=pltpu.CompilerParams(
            dimension_semantics=("arbitrary",),
        ),
    )(x, pbf, cn)
    return (preds, xhv)


# speedup vs baseline: 2.0131x; 1.6987x over previous
import jax
import jax.numpy as jnp
from jax.experimental import pallas as pl
from jax.experimental.pallas import tpu as pltpu


def _proj_prep_kernel(p_ref, pbf_ref, g_ref):
    p = p_ref[...].astype(jnp.bfloat16)
    pbf_ref[...] = p
    g_ref[...] = jax.lax.dot_general(
        p, p, (((0,), (0,)), ((), ())),
        preferred_element_type=jnp.float32).astype(jnp.bfloat16)


def _centroid_prep_kernel(c_ref, pbf_ref, m_ref):
    c = c_ref[...]
    s = jnp.sum(c * c, axis=1, keepdims=True)
    cn = (c * (1.0 / (jnp.sqrt(s) + 1e-12))).astype(jnp.bfloat16)
    m_ref[...] = jax.lax.dot_general(
        cn, pbf_ref[...], (((1,), (0,)), ((), ())),
        preferred_element_type=jnp.float32).astype(jnp.bfloat16)


def _main_kernel(x_ref, pbf_ref, m_ref, g_ref, xhv_ref, preds_ref):
    xf = x_ref[...]
    xb = xf.astype(jnp.bfloat16)
    xhv_ref[...] = jax.lax.dot_general(
        xb, pbf_ref[...], (((1,), (1,)), ((), ())),
        preferred_element_type=jnp.float32)
    t = jax.lax.dot_general(
        xb, g_ref[...], (((1,), (1,)), ((), ())),
        preferred_element_type=jnp.float32)
    s = jnp.sum(t * xf, axis=1, keepdims=True)
    factor = 1.0 / (jnp.sqrt(s) + 1e-12)
    p = jax.lax.dot_general(
        xb, m_ref[...], (((1,), (1,)), ((), ())),
        preferred_element_type=jnp.float32)
    preds_ref[...] = p * factor


@jax.jit
def kernel(x, projection, centroids):
    B, F = x.shape
    D, _ = projection.shape
    C, _ = centroids.shape

    pbf, g = pl.pallas_call(
        _proj_prep_kernel,
        in_specs=[pl.BlockSpec((D, F), lambda: (0, 0))],
        out_specs=[
            pl.BlockSpec((D, F), lambda: (0, 0)),
            pl.BlockSpec((F, F), lambda: (0, 0)),
        ],
        out_shape=[
            jax.ShapeDtypeStruct((D, F), jnp.bfloat16),
            jax.ShapeDtypeStruct((F, F), jnp.bfloat16),
        ],
    )(projection)

    BC = 256
    m = pl.pallas_call(
        _centroid_prep_kernel,
        grid=(C // BC,),
        in_specs=[
            pl.BlockSpec((BC, D), lambda i: (i, 0)),
            pl.BlockSpec((D, F), lambda i: (0, 0)),
        ],
        out_specs=pl.BlockSpec((BC, F), lambda i: (i, 0)),
        out_shape=jax.ShapeDtypeStruct((C, F), jnp.bfloat16),
    )(centroids, pbf)

    BI = 512
    xhv, preds = pl.pallas_call(
        _main_kernel,
        grid=(B // BI,),
        in_specs=[
            pl.BlockSpec((BI, F), lambda i: (i, 0)),
            pl.BlockSpec((D, F), lambda i: (0, 0)),
            pl.BlockSpec((C, F), lambda i: (0, 0)),
            pl.BlockSpec((F, F), lambda i: (0, 0)),
        ],
        out_specs=[
            pl.BlockSpec((BI, D), lambda i: (i, 0)),
            pl.BlockSpec((BI, C), lambda i: (i, 0)),
        ],
        out_shape=[
            jax.ShapeDtypeStruct((B, D), jnp.float32),
            jax.ShapeDtypeStruct((B, C), jnp.float32),
        ],
        compiler_params=pltpu.CompilerParams(
            dimension_semantics=("arbitrary",),
        ),
    )(x, pbf, m, g)
    return (preds, xhv)
